# baseline (speedup 1.0000x reference)
import functools

import numpy as np
import jax
import jax.numpy as jnp
from jax import lax
from jax.experimental import pallas as pl
from jax.experimental.pallas import tpu as pltpu
from jax.experimental.pallas import tpu_sc as plsc

_MAX_SEQ = 4096


def _pe_table(max_len, d):
    pos = np.arange(max_len, dtype=np.float64)[:, None]
    i = np.arange(d, dtype=np.float64)[None, :]
    angle_rates = 1.0 / np.power(10000.0, (2.0 * (i // 2)) / d)
    angles = pos * angle_rates
    pe = np.zeros((max_len, d), dtype=np.float64)
    pe[:, 0::2] = np.sin(angles[:, 0::2])
    pe[:, 1::2] = np.cos(angles[:, 1::2])
    return jnp.asarray(pe, dtype=jnp.float32)


_NC = 2
_NS = 16
_NW = _NC * _NS
_LANES = 16


@functools.partial(jax.jit, static_argnames=("seq",))
def _embed(idx_flat, token_table, pe, seq):
    n = idx_flat.shape[0]
    d = token_table.shape[1]
    rows_per_w = n // _NW
    chunk = 64
    n_chunks = rows_per_w // chunk
    mesh = plsc.VectorSubcoreMesh(core_axis_name="c", subcore_axis_name="s")

    @functools.partial(
        pl.kernel,
        mesh=mesh,
        out_type=jax.ShapeDtypeStruct((n, d), jnp.float32),
        scratch_types=[
            pltpu.VMEM((rows_per_w,), jnp.int32),
            pltpu.VMEM((chunk, d), jnp.float32),
            pltpu.VMEM((chunk, d), jnp.float32),
            pltpu.SemaphoreType.DMA,
        ],
    )
    def k(idx_hbm, table_hbm, pe_hbm, out_hbm, idx_v, rows_v, pe_v, sem):
        wid = lax.axis_index("s") * _NC + lax.axis_index("c")
        base = wid * rows_per_w
        pe_base = lax.rem(base, seq)
        pltpu.sync_copy(idx_hbm.at[pl.ds(base, rows_per_w)], idx_v)

        @pl.loop(0, n_chunks)
        def _(ci):
            off = ci * chunk
            gat = pltpu.async_copy(
                table_hbm.at[idx_v.at[pl.ds(off, chunk)]], rows_v, sem
            )
            pltpu.sync_copy(pe_hbm.at[pl.ds(pe_base + off, chunk)], pe_v)
            gat.wait()

            @pl.loop(0, chunk)
            def _(r):
                @pl.loop(0, d, step=_LANES)
                def _(c0):
                    rows_v[r, pl.ds(c0, _LANES)] = (
                        rows_v[r, pl.ds(c0, _LANES)] + pe_v[r, pl.ds(c0, _LANES)]
                    )

            pltpu.sync_copy(rows_v, out_hbm.at[pl.ds(base + off, chunk)])

    return k(idx_flat, token_table, pe)


def kernel(x, token_table):
    b, s = x.shape
    d = token_table.shape[1]
    pe = _pe_table(_MAX_SEQ, d)[:s]
    out = _embed(x.reshape(-1), token_table, pe, s)
    return out.reshape(b, s, d)


# speedup vs baseline: 1.7995x; 1.7995x over previous
import functools

import numpy as np
import jax
import jax.numpy as jnp
from jax import lax
from jax.experimental import pallas as pl
from jax.experimental.pallas import tpu as pltpu
from jax.experimental.pallas import tpu_sc as plsc

_MAX_SEQ = 4096


def _pe_table(max_len, d):
    pos = np.arange(max_len, dtype=np.float64)[:, None]
    i = np.arange(d, dtype=np.float64)[None, :]
    angle_rates = 1.0 / np.power(10000.0, (2.0 * (i // 2)) / d)
    angles = pos * angle_rates
    pe = np.zeros((max_len, d), dtype=np.float64)
    pe[:, 0::2] = np.sin(angles[:, 0::2])
    pe[:, 1::2] = np.cos(angles[:, 1::2])
    return jnp.asarray(pe, dtype=jnp.float32)


_NC = 2
_NS = 16
_NW = _NC * _NS
_LANES = 16
_NBUF = 3


@functools.partial(jax.jit, static_argnames=("batch", "seq"))
def _embed(idx_flat, token_table, pe, batch, seq):
    d = token_table.shape[1]
    n = idx_flat.shape[0]
    p_per_w = seq // _NW
    rows_per_w = p_per_w * batch
    half = p_per_w // 2
    nvec = d // _LANES
    chunks = [(b, h) for b in range(batch) for h in range(2)]
    mesh = plsc.VectorSubcoreMesh(core_axis_name="c", subcore_axis_name="s")

    @functools.partial(
        pl.kernel,
        mesh=mesh,
        out_type=jax.ShapeDtypeStruct((n, d), jnp.float32),
        scratch_types=[
            pltpu.VMEM((rows_per_w,), jnp.int32),
            pltpu.VMEM((p_per_w, d), jnp.float32),
            [pltpu.VMEM((half, d), jnp.float32) for _ in range(_NBUF)],
            [pltpu.SemaphoreType.DMA for _ in range(_NBUF)],
            [pltpu.SemaphoreType.DMA for _ in range(_NBUF)],
        ],
    )
    def k(idx_hbm, table_hbm, pe_hbm, out_hbm, idx_v, pe_v, bufs, gsems, wsems):
        wid = lax.axis_index("s") * _NC + lax.axis_index("c")
        pbase = wid * p_per_w

        for b in range(batch):
            pltpu.sync_copy(
                idx_hbm.at[pl.ds(b * seq + pbase, p_per_w)],
                idx_v.at[pl.ds(b * p_per_w, p_per_w)],
            )
        pltpu.sync_copy(pe_hbm.at[pl.ds(pbase, p_per_w)], pe_v)

        def start_gather(ci):
            b, h = chunks[ci]
            bi = ci % _NBUF
            return pltpu.async_copy(
                table_hbm.at[idx_v.at[pl.ds(b * p_per_w + h * half, half)]],
                bufs[bi],
                gsems[bi],
            )

        gh = start_gather(0)
        wb = [None] * _NBUF
        for ci, (b, h) in enumerate(chunks):
            bi = ci % _NBUF
            buf = bufs[bi]
            nci = ci + 1
            ngh = None
            if nci < len(chunks):
                nbi = nci % _NBUF
                if wb[nbi] is not None:
                    wb[nbi].wait()
                    wb[nbi] = None
                ngh = start_gather(nci)
            gh.wait()
            gh = ngh

            @pl.loop(0, half)
            def _(r):
                for c in range(nvec):
                    plsc.addupdate(
                        buf.at[r, pl.ds(c * _LANES, _LANES)],
                        pe_v[h * half + r, pl.ds(c * _LANES, _LANES)],
                    )

            wb[bi] = pltpu.async_copy(
                buf,
                out_hbm.at[pl.ds(b * seq + pbase + h * half, half)],
                wsems[bi],
            )
        for h_ in wb:
            if h_ is not None:
                h_.wait()

    return k(idx_flat, token_table, pe)


def kernel(x, token_table):
    b, s = x.shape
    d = token_table.shape[1]
    pe = _pe_table(_MAX_SEQ, d)[:s]
    out = _embed(x.reshape(-1), token_table, pe, b, s)
    return out.reshape(b, s, d)
